# baseline (speedup 1.0000x reference)
import functools

import jax
import jax.numpy as jnp
from jax import lax
from jax.experimental import pallas as pl
from jax.experimental.pallas import tpu as pltpu
from jax.experimental.pallas import tpu_sc as plsc

NC = 2
NS = 16
IW = 128
WZ = 624


def _split_copy(src, dst, s, n_rows):
  tail = n_rows - NS * WZ
  pltpu.sync_copy(src.at[pl.ds(s * WZ, WZ)], dst.at[pl.ds(s * WZ, WZ)])
  if tail:
    @pl.when(s == 0)
    def _():
      pltpu.sync_copy(src.at[pl.ds(NS * WZ, tail)],
                      dst.at[pl.ds(NS * WZ, tail)])


def _sc_segsum(fetch_list, idx2_list, zeros_h, n_dst, n_pad, width):
  nb = idx2_list[0][1].shape[0]
  bpt = nb // NS
  n_phases = len(fetch_list[0])
  wz_acc = n_pad // NS

  n_out = NC * n_phases
  out_type = (jax.ShapeDtypeStruct((n_dst, width), jnp.float32),) * n_out

  data_args = []
  layout = []
  for c in range(NC):
    ph_idx = []
    for kind, arr in fetch_list[c]:
      data_args.append(arr)
      ph_idx.append(len(data_args) - 1)
    src2, dst2 = idx2_list[c]
    s_i = None
    if src2 is not None:
      data_args.append(src2)
      s_i = len(data_args) - 1
    data_args.append(dst2)
    layout.append((ph_idx, s_i, len(data_args) - 1))
  data_args.append(zeros_h)
  z_i = len(data_args) - 1

  @functools.partial(
      pl.kernel,
      mesh=plsc.VectorSubcoreMesh(core_axis_name="c", subcore_axis_name="s"),
      out_type=out_type,
      scratch_types=[
          pltpu.VMEM((bpt, IW), jnp.int32),
          pltpu.VMEM((IW,), jnp.int32),
          pltpu.VMEM((IW,), jnp.int32),
          pltpu.VMEM((IW, width), jnp.float32),
          pltpu.VMEM((IW, width), jnp.float32),
          pltpu.SemaphoreType.DMA,
          pltpu.SemaphoreType.DMA,
          pltpu.SemaphoreType.DMA,
          pltpu.SemaphoreType.DMA,
          pltpu.SemaphoreType.DMA,
          pltpu.SemaphoreType.DMA,
          pltpu.VMEM_SHARED((n_pad, width), jnp.float32),
      ],
  )
  def k(*refs):
    args = refs[:len(data_args)]
    outs = refs[len(data_args):len(data_args) + n_out]
    (didx, sidx0, sidx1, rows0, rows1,
     si0, si1, sg0, sg1, ss0, ss1, acc) = refs[len(data_args) + n_out:]
    z_h = args[z_i]
    c = lax.axis_index("c")
    s = lax.axis_index("s")

    def run_core(core, ph_idx, s_i, d_i):
      pltpu.sync_copy(args[d_i].at[pl.ds(s * bpt, bpt)], didx)
      src2 = args[s_i] if s_i is not None else None

      for p in range(n_phases):
        kind, _ = fetch_list[core][p]
        src_h = args[ph_idx[p]]
        out_h = outs[core * n_phases + p]

        def idx_load(b, sidx, sem):
          pltpu.async_copy(src2.at[pl.ds((s * bpt + b) * IW, IW)], sidx, sem)

        def idx_wait(sidx, sem):
          pltpu.make_async_copy(src2.at[pl.ds(0, IW)], sidx, sem).wait()

        def fetch(b, sidx, rows, sem):
          if kind == "gather":
            pltpu.async_copy(src_h.at[sidx], rows, sem)
          else:
            pltpu.async_copy(src_h.at[pl.ds((s * bpt + b) * IW, IW)],
                             rows, sem)

        def fetch_wait(sidx, rows, sem):
          pltpu.make_async_copy(src_h.at[sidx] if kind == "gather"
                                else src_h.at[pl.ds(0, IW)],
                                rows, sem).wait()

        def scatter(b, rows, sem):
          pltpu.async_copy(rows, acc.at[didx.at[b]], sem, add=True)

        def scatter_wait(rows, sem):
          pltpu.make_async_copy(rows, acc.at[didx.at[0]], sem).wait()

        pltpu.sync_copy(z_h.at[pl.ds(s * wz_acc, wz_acc)],
                        acc.at[pl.ds(s * wz_acc, wz_acc)])
        plsc.subcore_barrier()

        if kind == "gather":
          idx_load(0, sidx0, si0)
          idx_load(1, sidx1, si1)
          idx_wait(sidx0, si0)
          fetch(0, sidx0, rows0, sg0)
          idx_wait(sidx1, si1)
          fetch(1, sidx1, rows1, sg1)
        else:
          fetch(0, sidx0, rows0, sg0)
          fetch(1, sidx1, rows1, sg1)

        def body(i, carry):
          b0 = 2 * i
          b1 = b0 + 1
          fetch_wait(sidx0, rows0, sg0)
          scatter(b0, rows0, ss0)
          if kind == "gather":
            idx_load(b0 + 2, sidx0, si0)
          fetch_wait(sidx1, rows1, sg1)
          scatter(b1, rows1, ss1)
          if kind == "gather":
            idx_load(b1 + 2, sidx1, si1)
          scatter_wait(rows0, ss0)
          if kind == "gather":
            idx_wait(sidx0, si0)
          fetch(b0 + 2, sidx0, rows0, sg0)
          scatter_wait(rows1, ss1)
          if kind == "gather":
            idx_wait(sidx1, si1)
          fetch(b1 + 2, sidx1, rows1, sg1)
          return carry

        lax.fori_loop(0, bpt // 2 - 1, body, 0)

        fetch_wait(sidx0, rows0, sg0)
        scatter(bpt - 2, rows0, ss0)
        fetch_wait(sidx1, rows1, sg1)
        scatter(bpt - 1, rows1, ss1)
        scatter_wait(rows0, ss0)
        scatter_wait(rows1, ss1)

        plsc.subcore_barrier()
        _split_copy(acc, out_h, s, n_dst)
        plsc.subcore_barrier()

    for core in range(NC):
      ph_idx, s_i, d_i = layout[core]

      @pl.when(c == core)
      def _():
        run_core(core, ph_idx, s_i, d_i)

  return k(*data_args)


def _tc_epilogue(seg0, seg1, a, h0, h1, we_aug, wl, wr, b, g, bt, last):
  n = seg0.shape[0]
  blk = 2000
  grid = (n // blk,)

  def body(seg0_r, seg1_r, a_r, h0_r, h1_r, wea_r, wl_r, wr_r, b_r,
           g_r, bt_r, *outs):
    av = a_r[...]
    cnt = jnp.maximum(av[:, 16:17], 1.0)
    ea = jnp.dot(av, wea_r[...], preferred_element_type=jnp.float32,
                 precision=lax.Precision.HIGHEST)
    seg = jnp.concatenate([seg0_r[...], seg1_r[...]], axis=1) + ea
    agg = seg / cnt
    h = jnp.concatenate([h0_r[...], h1_r[...]], axis=1)
    y = (jnp.dot(agg, wl_r[...], preferred_element_type=jnp.float32,
                 precision=lax.Precision.HIGHEST)
         + jnp.dot(h, wr_r[...], preferred_element_type=jnp.float32,
                   precision=lax.Precision.HIGHEST)
         + b_r[...])
    mu = jnp.mean(y, axis=1, keepdims=True)
    var = jnp.mean((y - mu) ** 2, axis=1, keepdims=True)
    z = (y - mu) * lax.rsqrt(var + 1e-5) * g_r[...] + bt_r[...]
    if last:
      outs[0][...] = z
    else:
      outs[0][...] = z[:, :128]
      outs[1][...] = z[:, 128:]

  row_spec = lambda w: pl.BlockSpec((blk, w), lambda i: (i, 0))
  full_spec = lambda r, w: pl.BlockSpec((r, w), lambda i: (0, 0))
  in_specs = [row_spec(128), row_spec(128), row_spec(128), row_spec(128),
              row_spec(128), full_spec(128, 256), full_spec(256, 256),
              full_spec(256, 256), full_spec(1, 256), full_spec(1, 256),
              full_spec(1, 256)]
  if last:
    out_shape = jax.ShapeDtypeStruct((n, 256), jnp.float32)
    out_specs = row_spec(256)
  else:
    out_shape = (jax.ShapeDtypeStruct((n, 128), jnp.float32),) * 2
    out_specs = (row_spec(128), row_spec(128))

  return pl.pallas_call(
      body, grid=grid, in_specs=in_specs, out_specs=out_specs,
      out_shape=out_shape,
  )(seg0, seg1, a, h0, h1, we_aug, wl, wr, b.reshape(1, 256),
    g.reshape(1, 256), bt.reshape(1, 256))


def kernel(x_user, x_item, edge_index_user_buys_item,
           edge_index_item_rev_buys_user, edge_time_user_buys_item,
           edge_time_item_rev_buys_user, We, be, Wl, bl, Wr, br,
           gamma, beta):
  n_user, d = x_user.shape
  n_item = x_item.shape[0]
  e = edge_time_user_buys_item.shape[0]
  layers = Wl.shape[0]
  assert n_user == n_item and d == 256

  e2 = ((e + 2 * NS * IW - 1) // (2 * NS * IW)) * (2 * NS * IW)
  n_pad = ((n_item + NS * 8 - 1) // (NS * 8)) * (NS * 8)
  dump = n_pad - 1
  epad = e2 - e

  def pad_idx(a, fill):
    return jnp.concatenate([a, jnp.full((epad,), fill, a.dtype)])

  src1_ui = pad_idx(edge_index_user_buys_item[0], 0)
  dst2_ui = pad_idx(edge_index_user_buys_item[1], dump).reshape(-1, IW)
  src1_iu = pad_idx(edge_index_item_rev_buys_user[0], 0)
  dst2_iu = pad_idx(edge_index_item_rev_buys_user[1], dump).reshape(-1, IW)

  def pad_et(et):
    block = jnp.concatenate([et, jnp.ones((e, 1), jnp.float32)], axis=1)
    return jnp.pad(block, ((0, epad), (0, IW - 17)))

  et_ui = pad_et(edge_time_user_buys_item)
  et_iu = pad_et(edge_time_item_rev_buys_user)

  zeros128 = jnp.zeros((n_pad, 128), jnp.float32)

  a_ui, a_iu = _sc_segsum(
      [[("linear", et_ui)], [("linear", et_iu)]],
      [(None, dst2_ui), (None, dst2_iu)],
      zeros128, n_item, n_pad, 128)

  zpad = jnp.zeros((111, 256), jnp.float32)
  we_aug0 = jnp.concatenate([We[0], be[0][None, :], zpad], axis=0)
  we_aug1 = jnp.concatenate([We[1], be[1][None, :], zpad], axis=0)

  h_u0, h_u1 = x_user[:, :128], x_user[:, 128:]
  h_i0, h_i1 = x_item[:, :128], x_item[:, 128:]

  for l in range(layers):
    seg_i0, seg_i1, seg_u0, seg_u1 = _sc_segsum(
        [[("gather", h_u0), ("gather", h_u1)],
         [("gather", h_i0), ("gather", h_i1)]],
        [(src1_ui, dst2_ui), (src1_iu, dst2_iu)],
        zeros128, n_item, n_pad, 128)
    last = l == layers - 1
    out_i = _tc_epilogue(seg_i0, seg_i1, a_ui, h_i0, h_i1, we_aug0,
                         Wl[l, 0], Wr[l, 0], bl[l, 0] + br[l, 0],
                         gamma[1], beta[1], last)
    out_u = _tc_epilogue(seg_u0, seg_u1, a_iu, h_u0, h_u1, we_aug1,
                         Wl[l, 1], Wr[l, 1], bl[l, 1] + br[l, 1],
                         gamma[0], beta[0], last)
    if last:
      return out_u, out_i
    h_i0, h_i1 = out_i
    h_u0, h_u1 = out_u


# speedup vs baseline: 1.0519x; 1.0519x over previous
import functools

import jax
import jax.numpy as jnp
from jax import lax
from jax.experimental import pallas as pl
from jax.experimental.pallas import tpu as pltpu
from jax.experimental.pallas import tpu_sc as plsc

NC = 2
NS = 16
IW = 128
WZ = 624


def _split_copy(src, dst, s, n_rows):
  tail = n_rows - NS * WZ
  pltpu.sync_copy(src.at[pl.ds(s * WZ, WZ)], dst.at[pl.ds(s * WZ, WZ)])
  if tail:
    @pl.when(s == 0)
    def _():
      pltpu.sync_copy(src.at[pl.ds(NS * WZ, tail)],
                      dst.at[pl.ds(NS * WZ, tail)])


def _sc_segsum(fetch_list, idx2_list, zeros_h, n_dst, n_pad, width):
  nb = idx2_list[0][1].shape[0]
  bpt = nb // NS
  n_phases = len(fetch_list[0])
  wz_acc = n_pad // NS

  n_out = NC * n_phases
  out_type = (jax.ShapeDtypeStruct((n_dst, width), jnp.float32),) * n_out

  data_args = []
  layout = []
  for c in range(NC):
    ph_idx = []
    for kind, arr in fetch_list[c]:
      data_args.append(arr)
      ph_idx.append(len(data_args) - 1)
    src2, dst2 = idx2_list[c]
    s_i = None
    if src2 is not None:
      data_args.append(src2)
      s_i = len(data_args) - 1
    data_args.append(dst2)
    layout.append((ph_idx, s_i, len(data_args) - 1))
  data_args.append(zeros_h)
  z_i = len(data_args) - 1

  @functools.partial(
      pl.kernel,
      mesh=plsc.VectorSubcoreMesh(core_axis_name="c", subcore_axis_name="s"),
      out_type=out_type,
      scratch_types=[
          pltpu.VMEM((bpt, IW), jnp.int32),
          pltpu.VMEM((IW,), jnp.int32),
          pltpu.VMEM((IW,), jnp.int32),
          pltpu.VMEM((IW, width), jnp.float32),
          pltpu.VMEM((IW, width), jnp.float32),
          pltpu.SemaphoreType.DMA,
          pltpu.SemaphoreType.DMA,
          pltpu.SemaphoreType.DMA,
          pltpu.SemaphoreType.DMA,
          pltpu.SemaphoreType.DMA,
          pltpu.SemaphoreType.DMA,
          pltpu.VMEM_SHARED((n_pad, width), jnp.float32),
      ],
  )
  def k(*refs):
    args = refs[:len(data_args)]
    outs = refs[len(data_args):len(data_args) + n_out]
    (didx, sidx0, sidx1, rows0, rows1,
     si0, si1, sg0, sg1, ss0, ss1, acc) = refs[len(data_args) + n_out:]
    z_h = args[z_i]
    c = lax.axis_index("c")
    s = lax.axis_index("s")

    def run_core(core, ph_idx, s_i, d_i):
      pltpu.sync_copy(args[d_i].at[pl.ds(s * bpt, bpt)], didx)
      src2 = args[s_i] if s_i is not None else None

      for p in range(n_phases):
        kind, _ = fetch_list[core][p]
        src_h = args[ph_idx[p]]
        out_h = outs[core * n_phases + p]

        def idx_load(b, sidx, sem):
          pltpu.async_copy(src2.at[pl.ds((s * bpt + b) * IW, IW)], sidx, sem)

        def idx_wait(sidx, sem):
          pltpu.make_async_copy(src2.at[pl.ds(0, IW)], sidx, sem).wait()

        def fetch(b, sidx, rows, sem):
          if kind == "gather":
            pltpu.async_copy(src_h.at[sidx], rows, sem)
          else:
            pltpu.async_copy(src_h.at[pl.ds((s * bpt + b) * IW, IW)],
                             rows, sem)

        def fetch_wait(sidx, rows, sem):
          pltpu.make_async_copy(src_h.at[sidx] if kind == "gather"
                                else src_h.at[pl.ds(0, IW)],
                                rows, sem).wait()

        def scatter(b, rows, sem):
          pltpu.async_copy(rows, acc.at[didx.at[b]], sem, add=True)

        def scatter_wait(rows, sem):
          pltpu.make_async_copy(rows, acc.at[didx.at[0]], sem).wait()

        pltpu.sync_copy(z_h.at[pl.ds(s * wz_acc, wz_acc)],
                        acc.at[pl.ds(s * wz_acc, wz_acc)])
        plsc.subcore_barrier()

        if kind == "gather":
          idx_load(0, sidx0, si0)
          idx_load(1, sidx1, si1)
          idx_wait(sidx0, si0)
          fetch(0, sidx0, rows0, sg0)
          idx_wait(sidx1, si1)
          fetch(1, sidx1, rows1, sg1)
        else:
          fetch(0, sidx0, rows0, sg0)
          fetch(1, sidx1, rows1, sg1)

        if kind == "gather":
          def body(i, carry):
            b0 = 2 * i
            b1 = b0 + 1
            fetch_wait(sidx0, rows0, sg0)
            idx_load(b0 + 2, sidx0, si0)
            pltpu.sync_copy(rows0, acc.at[didx.at[b0]], add=True)
            idx_wait(sidx0, si0)
            fetch(b0 + 2, sidx0, rows0, sg0)
            fetch_wait(sidx1, rows1, sg1)
            idx_load(b1 + 2, sidx1, si1)
            pltpu.sync_copy(rows1, acc.at[didx.at[b1]], add=True)
            idx_wait(sidx1, si1)
            fetch(b1 + 2, sidx1, rows1, sg1)
            return carry

          lax.fori_loop(0, bpt // 2 - 1, body, 0)
          fetch_wait(sidx0, rows0, sg0)
          pltpu.sync_copy(rows0, acc.at[didx.at[bpt - 2]], add=True)
          fetch_wait(sidx1, rows1, sg1)
          pltpu.sync_copy(rows1, acc.at[didx.at[bpt - 1]], add=True)
        else:
          def body(i, carry):
            b0 = 2 * i
            b1 = b0 + 1
            fetch_wait(sidx0, rows0, sg0)
            scatter(b0, rows0, ss0)
            fetch_wait(sidx1, rows1, sg1)
            scatter(b1, rows1, ss1)
            scatter_wait(rows0, ss0)
            fetch(b0 + 2, sidx0, rows0, sg0)
            scatter_wait(rows1, ss1)
            fetch(b1 + 2, sidx1, rows1, sg1)
            return carry

          lax.fori_loop(0, bpt // 2 - 1, body, 0)

          fetch_wait(sidx0, rows0, sg0)
          scatter(bpt - 2, rows0, ss0)
          fetch_wait(sidx1, rows1, sg1)
          scatter(bpt - 1, rows1, ss1)
          scatter_wait(rows0, ss0)
          scatter_wait(rows1, ss1)

        plsc.subcore_barrier()
        _split_copy(acc, out_h, s, n_dst)
        plsc.subcore_barrier()

    for core in range(NC):
      ph_idx, s_i, d_i = layout[core]

      @pl.when(c == core)
      def _():
        run_core(core, ph_idx, s_i, d_i)

  return k(*data_args)


def _tc_epilogue(seg0, seg1, a, h0, h1, we_aug, wl, wr, b, g, bt, last):
  n = seg0.shape[0]
  blk = 2000
  grid = (n // blk,)

  def body(seg0_r, seg1_r, a_r, h0_r, h1_r, wea_r, wl_r, wr_r, b_r,
           g_r, bt_r, *outs):
    av = a_r[...]
    cnt = jnp.maximum(av[:, 16:17], 1.0)
    ea = jnp.dot(av, wea_r[...], preferred_element_type=jnp.float32,
                 precision=lax.Precision.HIGHEST)
    seg = jnp.concatenate([seg0_r[...], seg1_r[...]], axis=1) + ea
    agg = seg / cnt
    h = jnp.concatenate([h0_r[...], h1_r[...]], axis=1)
    y = (jnp.dot(agg, wl_r[...], preferred_element_type=jnp.float32,
                 precision=lax.Precision.HIGHEST)
         + jnp.dot(h, wr_r[...], preferred_element_type=jnp.float32,
                   precision=lax.Precision.HIGHEST)
         + b_r[...])
    mu = jnp.mean(y, axis=1, keepdims=True)
    var = jnp.mean((y - mu) ** 2, axis=1, keepdims=True)
    z = (y - mu) * lax.rsqrt(var + 1e-5) * g_r[...] + bt_r[...]
    if last:
      outs[0][...] = z
    else:
      outs[0][...] = z[:, :128]
      outs[1][...] = z[:, 128:]

  row_spec = lambda w: pl.BlockSpec((blk, w), lambda i: (i, 0))
  full_spec = lambda r, w: pl.BlockSpec((r, w), lambda i: (0, 0))
  in_specs = [row_spec(128), row_spec(128), row_spec(128), row_spec(128),
              row_spec(128), full_spec(128, 256), full_spec(256, 256),
              full_spec(256, 256), full_spec(1, 256), full_spec(1, 256),
              full_spec(1, 256)]
  if last:
    out_shape = jax.ShapeDtypeStruct((n, 256), jnp.float32)
    out_specs = row_spec(256)
  else:
    out_shape = (jax.ShapeDtypeStruct((n, 128), jnp.float32),) * 2
    out_specs = (row_spec(128), row_spec(128))

  return pl.pallas_call(
      body, grid=grid, in_specs=in_specs, out_specs=out_specs,
      out_shape=out_shape,
  )(seg0, seg1, a, h0, h1, we_aug, wl, wr, b.reshape(1, 256),
    g.reshape(1, 256), bt.reshape(1, 256))


def kernel(x_user, x_item, edge_index_user_buys_item,
           edge_index_item_rev_buys_user, edge_time_user_buys_item,
           edge_time_item_rev_buys_user, We, be, Wl, bl, Wr, br,
           gamma, beta):
  n_user, d = x_user.shape
  n_item = x_item.shape[0]
  e = edge_time_user_buys_item.shape[0]
  layers = Wl.shape[0]
  assert n_user == n_item and d == 256

  e2 = ((e + 2 * NS * IW - 1) // (2 * NS * IW)) * (2 * NS * IW)
  n_pad = ((n_item + NS * 8 - 1) // (NS * 8)) * (NS * 8)
  dump = n_pad - 1
  epad = e2 - e

  def pad_idx(a, fill):
    return jnp.concatenate([a, jnp.full((epad,), fill, a.dtype)])

  src1_ui = pad_idx(edge_index_user_buys_item[0], 0)
  dst2_ui = pad_idx(edge_index_user_buys_item[1], dump).reshape(-1, IW)
  src1_iu = pad_idx(edge_index_item_rev_buys_user[0], 0)
  dst2_iu = pad_idx(edge_index_item_rev_buys_user[1], dump).reshape(-1, IW)

  def pad_et(et):
    block = jnp.concatenate([et, jnp.ones((e, 1), jnp.float32)], axis=1)
    return jnp.pad(block, ((0, epad), (0, IW - 17)))

  et_ui = pad_et(edge_time_user_buys_item)
  et_iu = pad_et(edge_time_item_rev_buys_user)

  zeros128 = jnp.zeros((n_pad, 128), jnp.float32)

  a_ui, a_iu = _sc_segsum(
      [[("linear", et_ui)], [("linear", et_iu)]],
      [(None, dst2_ui), (None, dst2_iu)],
      zeros128, n_item, n_pad, 128)

  zpad = jnp.zeros((111, 256), jnp.float32)
  we_aug0 = jnp.concatenate([We[0], be[0][None, :], zpad], axis=0)
  we_aug1 = jnp.concatenate([We[1], be[1][None, :], zpad], axis=0)

  h_u0, h_u1 = x_user[:, :128], x_user[:, 128:]
  h_i0, h_i1 = x_item[:, :128], x_item[:, 128:]

  for l in range(layers):
    seg_i0, seg_i1, seg_u0, seg_u1 = _sc_segsum(
        [[("gather", h_u0), ("gather", h_u1)],
         [("gather", h_i0), ("gather", h_i1)]],
        [(src1_ui, dst2_ui), (src1_iu, dst2_iu)],
        zeros128, n_item, n_pad, 128)
    last = l == layers - 1
    out_i = _tc_epilogue(seg_i0, seg_i1, a_ui, h_i0, h_i1, we_aug0,
                         Wl[l, 0], Wr[l, 0], bl[l, 0] + br[l, 0],
                         gamma[1], beta[1], last)
    out_u = _tc_epilogue(seg_u0, seg_u1, a_iu, h_u0, h_u1, we_aug1,
                         Wl[l, 1], Wr[l, 1], bl[l, 1] + br[l, 1],
                         gamma[0], beta[0], last)
    if last:
      return out_u, out_i
    h_i0, h_i1 = out_i
    h_u0, h_u1 = out_u
